# baseline (speedup 1.0000x reference)
import functools

import jax
import jax.numpy as jnp
from jax import lax
from jax.experimental import pallas as pl
from jax.experimental.pallas import tpu as pltpu
from jax.experimental.pallas import tpu_sc as plsc

N = 10000
E = 320000
NC = 2
NS = 16
NW = NC * NS
K = 80
NBUF = 4
EPT = 10240
EPAD = NW * EPT
N_CHUNKS = EPT // K
GC = 8
NGROUPS = N_CHUNKS // GC
GE = GC * K
NPAD = 10240
ROWS_PER_TILE = NPAD // NS
NZCOPY = ROWS_PER_TILE // K


def _mm1_body(x_ref, w_ref, o_ref):
    o_ref[...] = jnp.dot(x_ref[...], w_ref[...],
                         preferred_element_type=jnp.float32)


def _layer2_body(p_ref, b1_ref, w2_ref, o_ref):
    h = jnp.maximum(p_ref[0, :N, :] + p_ref[1, :N, :] + b1_ref[...], 0.0)
    o_ref[...] = jnp.dot(h, w2_ref[...], preferred_element_type=jnp.float32)


def _final_body(p_ref, b2_ref, o_ref):
    z = p_ref[0, :N, :] + p_ref[1, :N, :] + b2_ref[...]
    m = jnp.max(z, axis=1, keepdims=True)
    s = jnp.sum(jnp.exp(z - m), axis=1, keepdims=True)
    o_ref[...] = z - m - jnp.log(s)


@functools.lru_cache(maxsize=None)
def _make_spmm(D: int, SW: int):
    compact = SW != D
    mesh = plsc.VectorSubcoreMesh(core_axis_name="c", subcore_axis_name="s",
                                  num_cores=NC, num_subcores=NS)
    scratch = [
        pltpu.VMEM((2, GC, K), jnp.int32),
        pltpu.VMEM((2, GC, K), jnp.int32),
        pltpu.VMEM((GE,), jnp.float32),
        pltpu.VMEM((GE,), jnp.float32),
        pltpu.VMEM((NBUF, K, D), jnp.float32),
        pltpu.VMEM_SHARED((NPAD, SW), jnp.float32),
        pltpu.SemaphoreType.DMA,
        pltpu.SemaphoreType.DMA,
        pltpu.SemaphoreType.DMA,
        pltpu.SemaphoreType.DMA,
    ]
    if compact:
        scratch.append(pltpu.VMEM((2, K, SW), jnp.float32))

    @functools.partial(
        pl.kernel,
        out_type=jax.ShapeDtypeStruct((NC, NPAD, SW), jnp.float32),
        mesh=mesh,
        compiler_params=pltpu.CompilerParams(needs_layout_passes=False),
        scratch_types=scratch,
    )
    def spmm(sup, col, row, w, out, col_st, row_st, w_st0, w_st1, rows_v,
             acc, gsem, ssem, stsem, zsem, *maybe_srows):
        srows = maybe_srows[0] if compact else None
        w_sts = (w_st0, w_st1)
        cid = lax.axis_index("c")
        sid = lax.axis_index("s")
        wid = sid * NC + cid

        zero = jnp.zeros((16,), jnp.float32)
        zsrc = srows if compact else rows_v

        def zfill(r, carry):
            for j in range(SW // 16):
                zsrc[0, r, pl.ds(j * 16, 16)] = zero
            return carry

        lax.fori_loop(0, K, zfill, 0)
        zds = [
            pltpu.make_async_copy(
                zsrc.at[0], acc.at[pl.ds(sid * ROWS_PER_TILE + i * K, K)],
                zsem)
            for i in range(NZCOPY)
        ]
        for d in zds:
            d.start()

        pltpu.sync_copy(col.at[wid, 0], col_st.at[0])
        pltpu.sync_copy(row.at[wid, 0], row_st.at[0])
        pltpu.sync_copy(w.at[pl.ds(wid * EPT, GE)], w_st0)

        with jax.named_scope("prolog"):
            if compact:
                pltpu.async_copy(sup.at[col_st.at[0, 0]], rows_v.at[0], gsem)
                pltpu.async_copy(sup.at[col_st.at[0, 1]], rows_v.at[1], gsem)
                for d in zds:
                    d.wait()
            else:
                for d in zds:
                    d.wait()
                pltpu.async_copy(sup.at[col_st.at[0, 0]], rows_v.at[0], gsem)
                pltpu.async_copy(sup.at[col_st.at[0, 1]], rows_v.at[1], gsem)
            plsc.subcore_barrier()

        for g in range(NGROUPS):
            ib = g % 2
            nib = 1 - ib
            if g < NGROUPS - 1:
                std = [
                    pltpu.make_async_copy(col.at[wid, g + 1], col_st.at[nib],
                                          stsem),
                    pltpu.make_async_copy(row.at[wid, g + 1], row_st.at[nib],
                                          stsem),
                    pltpu.make_async_copy(
                        w.at[pl.ds(wid * EPT + (g + 1) * GE, GE)],
                        w_sts[nib], stsem),
                ]
                for d in std:
                    d.start()

            w_st = w_sts[ib]

            def chunk_body(cc, carry2, g=g, ib=ib, w_st=w_st):
                c = g * GC + cc
                b = lax.rem(c, NBUF)
                with jax.named_scope("gwait"):
                    pltpu.make_async_copy(sup.at[col_st.at[ib, cc]],
                                          rows_v.at[b], gsem).wait()

                sb = lax.rem(c, 2)

                def scale(q, inner):
                    e = q * 2
                    base = jnp.full((16,), 0, jnp.int32) + cc * K + e
                    wv0 = plsc.load_gather(w_st, [base])
                    wv1 = plsc.load_gather(w_st, [base + 1])
                    for j in range(SW // 16):
                        sl = pl.ds(j * 16, 16)
                        if compact:
                            srows[sb, e, sl] = rows_v[b, e, sl] * wv0
                            srows[sb, e + 1, sl] = rows_v[b, e + 1, sl] * wv1
                        else:
                            rows_v[b, e, sl] = rows_v[b, e, sl] * wv0
                            rows_v[b, e + 1, sl] = rows_v[b, e + 1, sl] * wv1
                    return inner

                with jax.named_scope("scale"):
                    lax.fori_loop(0, K // 2, scale, 0)
                ssrc = srows.at[sb] if compact else rows_v.at[b]
                with jax.named_scope("sissue"):
                    pltpu.async_copy(ssrc, acc.at[row_st.at[ib, cc]],
                                     ssem, add=True)

                @pl.when(c >= 1)
                def _():
                    with jax.named_scope("sabsorb"):
                        pltpu.make_async_copy(ssrc,
                                              acc.at[row_st.at[ib, cc]],
                                              ssem).wait()

                @pl.when(cc < GC - 2)
                def _():
                    with jax.named_scope("gissue"):
                        cnext = jnp.minimum(cc + 2, GC - 1)
                        bnext = lax.rem(c + 2, NBUF)
                        pltpu.async_copy(sup.at[col_st.at[ib, cnext]],
                                         rows_v.at[bnext], gsem)

                return carry2

            lax.fori_loop(0, GC, chunk_body, 0)

            if g < NGROUPS - 1:
                with jax.named_scope("stgwait"):
                    for d in std:
                        d.wait()
                    c0 = (g + 1) * GC
                    pltpu.async_copy(sup.at[col_st.at[nib, 0]],
                                     rows_v.at[c0 % NBUF], gsem)
                    pltpu.async_copy(sup.at[col_st.at[nib, 1]],
                                     rows_v.at[(c0 + 1) % NBUF], gsem)

        with jax.named_scope("findrain"):
            fsrc = srows.at[0] if compact else rows_v.at[0]
            pltpu.make_async_copy(fsrc, acc.at[row_st.at[0, 0]],
                                  ssem).wait()
        with jax.named_scope("finbar"):
            plsc.subcore_barrier()

        with jax.named_scope("wb"):
            pltpu.sync_copy(
                acc.at[pl.ds(sid * ROWS_PER_TILE, ROWS_PER_TILE)],
                out.at[cid, pl.ds(sid * ROWS_PER_TILE, ROWS_PER_TILE)])

    return spmm


def kernel(x, edge_index, edge_weight, W1, b1, W2, b2):
    pad = EPAD - E
    col = jnp.concatenate(
        [edge_index[1], jnp.zeros((pad,), jnp.int32)]).reshape(
            NW, NGROUPS, GC, K)
    row = jnp.concatenate(
        [edge_index[0], jnp.zeros((pad,), jnp.int32)]).reshape(
            NW, NGROUPS, GC, K)
    w3 = jnp.concatenate([edge_weight, jnp.zeros((pad,), jnp.float32)])

    support1 = pl.pallas_call(
        _mm1_body,
        out_shape=jax.ShapeDtypeStruct((N, 128), jnp.float32),
    )(x, W1)

    parts1 = _make_spmm(128, 128)(support1, col, row, w3)

    W2p = jnp.concatenate([W2, jnp.zeros((128, 64), jnp.float32)], axis=1)
    support2 = pl.pallas_call(
        _layer2_body,
        out_shape=jax.ShapeDtypeStruct((N, 128), jnp.float32),
    )(parts1, b1.reshape(1, 128), W2p)

    parts2 = _make_spmm(128, 64)(support2, col, row, w3)

    out = pl.pallas_call(
        _final_body,
        out_shape=jax.ShapeDtypeStruct((N, 64), jnp.float32),
    )(parts2, b2.reshape(1, 64))

    return out


# speedup vs baseline: 1.0241x; 1.0241x over previous
import functools

import jax
import jax.numpy as jnp
from jax import lax
from jax.experimental import pallas as pl
from jax.experimental.pallas import tpu as pltpu
from jax.experimental.pallas import tpu_sc as plsc

N = 10000
E = 320000
NC = 2
NS = 16
NW = NC * NS
K = 80
NBUF = 4
EPT = 10240
EPAD = NW * EPT
N_CHUNKS = EPT // K
GC = 8
NGROUPS = N_CHUNKS // GC
GE = GC * K
NPAD = 10240
ROWS_PER_TILE = NPAD // NS
NZCOPY = ROWS_PER_TILE // K


def _mm1_body(x_ref, w_ref, o_ref):
    o_ref[...] = jnp.dot(x_ref[...], w_ref[...],
                         preferred_element_type=jnp.float32)


def _layer2_body(p_ref, b1_ref, w2_ref, o_ref):
    h = jnp.maximum(p_ref[0, :N, :] + p_ref[1, :N, :] + b1_ref[...], 0.0)
    o_ref[...] = jnp.dot(h, w2_ref[...], preferred_element_type=jnp.float32)


def _final_body(p_ref, b2_ref, o_ref):
    z = p_ref[0, :N, :64] + p_ref[1, :N, :64] + b2_ref[...]
    m = jnp.max(z, axis=1, keepdims=True)
    s = jnp.sum(jnp.exp(z - m), axis=1, keepdims=True)
    o_ref[...] = z - m - jnp.log(s)


SCATTER_ON = False

@functools.lru_cache(maxsize=None)
def _make_spmm(D: int, SW: int):
    compact = SW != D
    mesh = plsc.VectorSubcoreMesh(core_axis_name="c", subcore_axis_name="s",
                                  num_cores=NC, num_subcores=NS)
    scratch = [
        pltpu.VMEM((2, GC, K), jnp.int32),
        pltpu.VMEM((2, GC, K), jnp.int32),
        pltpu.VMEM((GE,), jnp.float32),
        pltpu.VMEM((GE,), jnp.float32),
        pltpu.VMEM((NBUF, K, D), jnp.float32),
        pltpu.VMEM_SHARED((NPAD, SW), jnp.float32),
        pltpu.SemaphoreType.DMA,
        pltpu.SemaphoreType.DMA,
        pltpu.SemaphoreType.DMA,
        pltpu.SemaphoreType.DMA,
    ]
    if compact:
        scratch.append(pltpu.VMEM((2, K, SW), jnp.float32))

    @functools.partial(
        pl.kernel,
        out_type=jax.ShapeDtypeStruct((NC, NPAD, SW), jnp.float32),
        mesh=mesh,
        compiler_params=pltpu.CompilerParams(needs_layout_passes=False),
        scratch_types=scratch,
    )
    def spmm(sup, col, row, w, out, col_st, row_st, w_st0, w_st1, rows_v,
             acc, gsem, ssem, stsem, zsem, *maybe_srows):
        srows = maybe_srows[0] if compact else None
        w_sts = (w_st0, w_st1)
        cid = lax.axis_index("c")
        sid = lax.axis_index("s")
        wid = sid * NC + cid

        zero = jnp.zeros((16,), jnp.float32)
        zsrc = srows if compact else rows_v

        def zfill(r, carry):
            for j in range(SW // 16):
                zsrc[0, r, pl.ds(j * 16, 16)] = zero
            return carry

        lax.fori_loop(0, K, zfill, 0)
        zds = [
            pltpu.make_async_copy(
                zsrc.at[0], acc.at[pl.ds(sid * ROWS_PER_TILE + i * K, K)],
                zsem)
            for i in range(NZCOPY)
        ]
        for d in zds:
            d.start()

        pltpu.sync_copy(col.at[wid, 0], col_st.at[0])
        pltpu.sync_copy(row.at[wid, 0], row_st.at[0])
        pltpu.sync_copy(w.at[pl.ds(wid * EPT, GE)], w_st0)

        with jax.named_scope("prolog"):
            if compact:
                pltpu.async_copy(sup.at[col_st.at[0, 0]], rows_v.at[0], gsem)
                pltpu.async_copy(sup.at[col_st.at[0, 1]], rows_v.at[1], gsem)
                for d in zds:
                    d.wait()
            else:
                for d in zds:
                    d.wait()
                pltpu.async_copy(sup.at[col_st.at[0, 0]], rows_v.at[0], gsem)
                pltpu.async_copy(sup.at[col_st.at[0, 1]], rows_v.at[1], gsem)
            plsc.subcore_barrier()

        for g in range(NGROUPS):
            ib = g % 2
            nib = 1 - ib
            if g < NGROUPS - 1:
                std = [
                    pltpu.make_async_copy(col.at[wid, g + 1], col_st.at[nib],
                                          stsem),
                    pltpu.make_async_copy(row.at[wid, g + 1], row_st.at[nib],
                                          stsem),
                    pltpu.make_async_copy(
                        w.at[pl.ds(wid * EPT + (g + 1) * GE, GE)],
                        w_sts[nib], stsem),
                ]
                for d in std:
                    d.start()

            w_st = w_sts[ib]

            def chunk_body(cc, carry2, g=g, ib=ib, w_st=w_st):
                c = g * GC + cc
                b = lax.rem(c, NBUF)
                with jax.named_scope("gwait"):
                    pltpu.make_async_copy(sup.at[col_st.at[ib, cc]],
                                          rows_v.at[b], gsem).wait()

                sb = lax.rem(c, 2)

                def scale(q, inner):
                    e = q * 2
                    base = jnp.full((16,), 0, jnp.int32) + cc * K + e
                    wv0 = plsc.load_gather(w_st, [base])
                    wv1 = plsc.load_gather(w_st, [base + 1])
                    for j in range(SW // 16):
                        sl = pl.ds(j * 16, 16)
                        if compact:
                            srows[sb, e, sl] = rows_v[b, e, sl] * wv0
                            srows[sb, e + 1, sl] = rows_v[b, e + 1, sl] * wv1
                        else:
                            rows_v[b, e, sl] = rows_v[b, e, sl] * wv0
                            rows_v[b, e + 1, sl] = rows_v[b, e + 1, sl] * wv1
                    return inner

                with jax.named_scope("scale"):
                    lax.fori_loop(0, K // 2, scale, 0)
                ssrc = srows.at[sb] if compact else rows_v.at[b]
                if SCATTER_ON:
                    with jax.named_scope("sissue"):
                        pltpu.async_copy(ssrc, acc.at[row_st.at[ib, cc]],
                                         ssem, add=True)

                    @pl.when(c >= 1)
                    def _():
                        with jax.named_scope("sabsorb"):
                            pltpu.make_async_copy(ssrc,
                                                  acc.at[row_st.at[ib, cc]],
                                                  ssem).wait()

                @pl.when(cc < GC - 2)
                def _():
                    with jax.named_scope("gissue"):
                        cnext = jnp.minimum(cc + 2, GC - 1)
                        bnext = lax.rem(c + 2, NBUF)
                        pltpu.async_copy(sup.at[col_st.at[ib, cnext]],
                                         rows_v.at[bnext], gsem)

                return carry2

            lax.fori_loop(0, GC, chunk_body, 0)

            if g < NGROUPS - 1:
                with jax.named_scope("stgwait"):
                    for d in std:
                        d.wait()
                    c0 = (g + 1) * GC
                    pltpu.async_copy(sup.at[col_st.at[nib, 0]],
                                     rows_v.at[c0 % NBUF], gsem)
                    pltpu.async_copy(sup.at[col_st.at[nib, 1]],
                                     rows_v.at[(c0 + 1) % NBUF], gsem)

        if SCATTER_ON:
            with jax.named_scope("findrain"):
                fsrc = srows.at[0] if compact else rows_v.at[0]
                pltpu.make_async_copy(fsrc, acc.at[row_st.at[0, 0]],
                                      ssem).wait()
        with jax.named_scope("finbar"):
            plsc.subcore_barrier()

        with jax.named_scope("wb"):
            pltpu.sync_copy(
                acc.at[pl.ds(sid * ROWS_PER_TILE, ROWS_PER_TILE)],
                out.at[cid, pl.ds(sid * ROWS_PER_TILE, ROWS_PER_TILE)])

    return spmm


def kernel(x, edge_index, edge_weight, W1, b1, W2, b2):
    pad = EPAD - E
    col = jnp.concatenate(
        [edge_index[1], jnp.zeros((pad,), jnp.int32)]).reshape(
            NW, NGROUPS, GC, K)
    row = jnp.concatenate(
        [edge_index[0], jnp.zeros((pad,), jnp.int32)]).reshape(
            NW, NGROUPS, GC, K)
    w3 = jnp.concatenate([edge_weight, jnp.zeros((pad,), jnp.float32)])

    support1 = pl.pallas_call(
        _mm1_body,
        out_shape=jax.ShapeDtypeStruct((N, 128), jnp.float32),
    )(x, W1)

    parts1 = _make_spmm(128, 128)(support1, col, row, w3)

    W2p = jnp.concatenate([W2, jnp.zeros((128, 64), jnp.float32)], axis=1)
    support2 = pl.pallas_call(
        _layer2_body,
        out_shape=jax.ShapeDtypeStruct((N, 128), jnp.float32),
    )(parts1, b1.reshape(1, 128), W2p)

    parts2 = _make_spmm(128, 128)(support2, col, row, w3)

    out = pl.pallas_call(
        _final_body,
        out_shape=jax.ShapeDtypeStruct((N, 64), jnp.float32),
    )(parts2, b2.reshape(1, 64))

    return out


# speedup vs baseline: 3.5296x; 3.4466x over previous
import functools

import jax
import jax.numpy as jnp
from jax import lax
from jax.experimental import pallas as pl
from jax.experimental.pallas import tpu as pltpu
from jax.experimental.pallas import tpu_sc as plsc

N = 10000
E = 320000
NC = 2
NS = 16
NW = NC * NS
K = 80
NBUF = 4
EPT = 10240
EPAD = NW * EPT
N_CHUNKS = EPT // K
GC = 8
NGROUPS = N_CHUNKS // GC
GE = GC * K
NPAD = 10240
ROWS_PER_TILE = NPAD // NS
NZCOPY = ROWS_PER_TILE // K


def _mm1_body(x_ref, w_ref, o_ref):
    o_ref[...] = jnp.dot(x_ref[...], w_ref[...],
                         preferred_element_type=jnp.float32)


def _layer2_body(p_ref, b1_ref, w2_ref, o_ref):
    h = jnp.maximum(p_ref[0, :N, :] + p_ref[1, :N, :] + b1_ref[...], 0.0)
    o_ref[...] = jnp.dot(h, w2_ref[...], preferred_element_type=jnp.float32)


def _final_body(p_ref, b2_ref, o_ref):
    z = p_ref[0, :N, :64] + p_ref[1, :N, :64] + b2_ref[...]
    m = jnp.max(z, axis=1, keepdims=True)
    s = jnp.sum(jnp.exp(z - m), axis=1, keepdims=True)
    o_ref[...] = z - m - jnp.log(s)


SCATTER_ON = True

@functools.lru_cache(maxsize=None)
def _make_spmm(D: int, SW: int):
    compact = SW != D
    mesh = plsc.VectorSubcoreMesh(core_axis_name="c", subcore_axis_name="s",
                                  num_cores=NC, num_subcores=NS)
    scratch = [
        pltpu.VMEM((2, GC, K), jnp.int32),
        pltpu.VMEM((2, GC, K), jnp.int32),
        pltpu.VMEM((GE,), jnp.float32),
        pltpu.VMEM((GE,), jnp.float32),
        pltpu.VMEM((NBUF, K, D), jnp.float32),
        pltpu.VMEM_SHARED((NPAD, SW), jnp.float32),
        pltpu.SemaphoreType.DMA,
        pltpu.SemaphoreType.DMA,
        pltpu.SemaphoreType.DMA,
        pltpu.SemaphoreType.DMA,
    ]
    if compact:
        scratch.append(pltpu.VMEM((2, K, SW), jnp.float32))

    @functools.partial(
        pl.kernel,
        out_type=jax.ShapeDtypeStruct((NC, NPAD, SW), jnp.float32),
        mesh=mesh,
        compiler_params=pltpu.CompilerParams(needs_layout_passes=False),
        scratch_types=scratch,
    )
    def spmm(sup, col, row, w, out, col_st, row_st, w_st0, w_st1, rows_v,
             acc, gsem, ssem, stsem, zsem, *maybe_srows):
        srows = maybe_srows[0] if compact else None
        w_sts = (w_st0, w_st1)
        cid = lax.axis_index("c")
        sid = lax.axis_index("s")
        wid = sid * NC + cid

        zero = jnp.zeros((16,), jnp.float32)
        zsrc = srows if compact else rows_v

        def zfill(r, carry):
            for j in range(SW // 16):
                zsrc[0, r, pl.ds(j * 16, 16)] = zero
            return carry

        lax.fori_loop(0, K, zfill, 0)
        zds = [
            pltpu.make_async_copy(
                zsrc.at[0], acc.at[pl.ds(sid * ROWS_PER_TILE + i * K, K)],
                zsem)
            for i in range(NZCOPY)
        ]
        for d in zds:
            d.start()

        pltpu.sync_copy(col.at[wid, 0], col_st.at[0])
        pltpu.sync_copy(row.at[wid, 0], row_st.at[0])
        pltpu.sync_copy(w.at[pl.ds(wid * EPT, GE)], w_st0)

        with jax.named_scope("prolog"):
            if compact:
                pltpu.async_copy(sup.at[col_st.at[0, 0]], rows_v.at[0], gsem)
                pltpu.async_copy(sup.at[col_st.at[0, 1]], rows_v.at[1], gsem)
                for d in zds:
                    d.wait()
            else:
                for d in zds:
                    d.wait()
                pltpu.async_copy(sup.at[col_st.at[0, 0]], rows_v.at[0], gsem)
                pltpu.async_copy(sup.at[col_st.at[0, 1]], rows_v.at[1], gsem)
            plsc.subcore_barrier()

        for g in range(NGROUPS):
            ib = g % 2
            nib = 1 - ib
            if g < NGROUPS - 1:
                std = [
                    pltpu.make_async_copy(col.at[wid, g + 1], col_st.at[nib],
                                          stsem),
                    pltpu.make_async_copy(row.at[wid, g + 1], row_st.at[nib],
                                          stsem),
                    pltpu.make_async_copy(
                        w.at[pl.ds(wid * EPT + (g + 1) * GE, GE)],
                        w_sts[nib], stsem),
                ]
                for d in std:
                    d.start()

            w_st = w_sts[ib]

            def chunk_body(cc, carry2, g=g, ib=ib, w_st=w_st):
                c = g * GC + cc
                b = lax.rem(c, NBUF)
                with jax.named_scope("gwait"):
                    pltpu.make_async_copy(sup.at[col_st.at[ib, cc]],
                                          rows_v.at[b], gsem).wait()

                sb = lax.rem(c, 2)

                def scale(q, inner):
                    e = q * 2
                    base = jnp.full((16,), 0, jnp.int32) + cc * K + e
                    wv0 = plsc.load_gather(w_st, [base])
                    wv1 = plsc.load_gather(w_st, [base + 1])
                    for j in range(SW // 16):
                        sl = pl.ds(j * 16, 16)
                        if compact:
                            srows[sb, e, sl] = rows_v[b, e, sl] * wv0
                            srows[sb, e + 1, sl] = rows_v[b, e + 1, sl] * wv1
                        else:
                            rows_v[b, e, sl] = rows_v[b, e, sl] * wv0
                            rows_v[b, e + 1, sl] = rows_v[b, e + 1, sl] * wv1
                    return inner

                with jax.named_scope("scale"):
                    lax.fori_loop(0, K // 2, scale, 0)
                ssrc = srows.at[sb] if compact else rows_v.at[b]
                if SCATTER_ON:
                    with jax.named_scope("sissue"):
                        pltpu.async_copy(ssrc, acc.at[row_st.at[ib, cc]],
                                         ssem, add=True)

                    @pl.when(c >= 1)
                    def _():
                        with jax.named_scope("sabsorb"):
                            pltpu.make_async_copy(ssrc,
                                                  acc.at[row_st.at[ib, cc]],
                                                  ssem).wait()

                @pl.when(cc < GC - 2)
                def _():
                    with jax.named_scope("gissue"):
                        cnext = jnp.minimum(cc + 2, GC - 1)
                        bnext = lax.rem(c + 2, NBUF)
                        pltpu.async_copy(sup.at[col_st.at[ib, cnext]],
                                         rows_v.at[bnext], gsem)

                return carry2

            lax.fori_loop(0, GC, chunk_body, 0)

            if g < NGROUPS - 1:
                with jax.named_scope("stgwait"):
                    for d in std:
                        d.wait()
                    c0 = (g + 1) * GC
                    pltpu.async_copy(sup.at[col_st.at[nib, 0]],
                                     rows_v.at[c0 % NBUF], gsem)
                    pltpu.async_copy(sup.at[col_st.at[nib, 1]],
                                     rows_v.at[(c0 + 1) % NBUF], gsem)

        if SCATTER_ON:
            with jax.named_scope("findrain"):
                fsrc = srows.at[0] if compact else rows_v.at[0]
                pltpu.make_async_copy(fsrc, acc.at[row_st.at[0, 0]],
                                      ssem).wait()
        with jax.named_scope("finbar"):
            plsc.subcore_barrier()

        with jax.named_scope("wb"):
            pltpu.sync_copy(
                acc.at[pl.ds(sid * ROWS_PER_TILE, ROWS_PER_TILE)],
                out.at[cid, pl.ds(sid * ROWS_PER_TILE, ROWS_PER_TILE)])

    return spmm


def kernel(x, edge_index, edge_weight, W1, b1, W2, b2):
    pad = EPAD - E
    spread = jnp.arange(pad, dtype=jnp.int32) % N
    col = jnp.concatenate([edge_index[1], spread]).reshape(
        NW, NGROUPS, GC, K)
    row = jnp.concatenate([edge_index[0], spread]).reshape(
        NW, NGROUPS, GC, K)
    w3 = jnp.concatenate([edge_weight, jnp.zeros((pad,), jnp.float32)])

    support1 = pl.pallas_call(
        _mm1_body,
        out_shape=jax.ShapeDtypeStruct((N, 128), jnp.float32),
    )(x, W1)

    parts1 = _make_spmm(128, 128)(support1, col, row, w3)

    W2p = jnp.concatenate([W2, jnp.zeros((128, 64), jnp.float32)], axis=1)
    support2 = pl.pallas_call(
        _layer2_body,
        out_shape=jax.ShapeDtypeStruct((N, 128), jnp.float32),
    )(parts1, b1.reshape(1, 128), W2p)

    parts2 = _make_spmm(128, 128)(support2, col, row, w3)

    out = pl.pallas_call(
        _final_body,
        out_shape=jax.ShapeDtypeStruct((N, 64), jnp.float32),
    )(parts2, b2.reshape(1, 64))

    return out
